# baseline (speedup 1.0000x reference)
import jax
import jax.numpy as jnp
from jax.experimental import pallas as pl
from jax.experimental.pallas import tpu as pltpu

_SAMPLE = 8


def _pass_kernel(idx_ref, x_ref, y_ref, xs_ref, ysum_ref, acc_ref):
    step = pl.program_id(0)
    nsteps = pl.num_programs(0)
    rows = x_ref.shape[0]

    @pl.when(step == 0)
    def _init():
        acc_ref[0] = 0.0
        acc_ref[1] = 0.0
        ysum_ref[...] = jnp.zeros_like(ysum_ref)

    x = x_ref[...]
    acc_ref[0] += jnp.sum(x)
    acc_ref[1] += jnp.sum(x * x)

    base = step * rows
    for j in range(_SAMPLE):
        idx = idx_ref[j]
        local = idx - base

        @pl.when((idx >= base) & (idx < base + rows))
        def _copy():
            xs_ref[pl.ds(j, 1), :] = x_ref[pl.ds(local, 1), :]
            ysum_ref[...] += y_ref[pl.ds(local, 1), :]

    @pl.when(step == nsteps - 1)
    def _final():
        total = jnp.float32(x_ref.shape[1] * rows) * jnp.float32(nsteps)
        mean = acc_ref[0] / total
        var = acc_ref[1] / total - mean * mean
        inv_std = jax.lax.rsqrt(var)
        xs_ref[...] = (xs_ref[...] - mean) * inv_std


def kernel(x_train, y_train, indices):
    n, h, w = x_train.shape
    f = h * w
    x2 = x_train.reshape(n, f)
    y2 = y_train.reshape(n, 1)
    rows = 3000
    assert n % rows == 0
    grid = n // rows

    xs, ysum = pl.pallas_call(
        _pass_kernel,
        grid=(grid,),
        in_specs=[
            pl.BlockSpec(memory_space=pltpu.SMEM),
            pl.BlockSpec((rows, f), lambda i: (i, 0)),
            pl.BlockSpec((rows, 1), lambda i: (i, 0)),
        ],
        out_specs=[
            pl.BlockSpec((_SAMPLE, f), lambda i: (0, 0)),
            pl.BlockSpec((1, 1), lambda i: (0, 0)),
        ],
        out_shape=[
            jax.ShapeDtypeStruct((_SAMPLE, f), jnp.float32),
            jax.ShapeDtypeStruct((1, 1), y_train.dtype),
        ],
        scratch_shapes=[pltpu.SMEM((2,), jnp.float32)],
    )(indices, x2, y2)
    return xs.reshape(_SAMPLE, h, w), ysum[0, 0]
